# baseline (speedup 1.0000x reference)
import functools

import jax
import jax.numpy as jnp
from jax import lax
from jax.experimental import pallas as pl
from jax.experimental.pallas import tpu as pltpu
from jax.experimental.pallas import tpu_sc as plsc

N_NODES = 100000
D_FEAT = 128
NUM_GRAPHS = 512

CHUNK = 400
SUB = 80
NSUB = CHUNK // SUB
NCHUNKS = N_NODES // CHUNK
NTILES = 16
NCORES = 2
NWORKERS = NCORES * NTILES
NITER = (NCHUNKS + NWORKERS - 1) // NWORKERS
SEG_PER_TILE = NUM_GRAPHS // NTILES


def _sc_body(x_hbm, bflat_hbm, psum_hbm, pcnt_hbm, sed_hbm, idx_v,
             rows_v, idx1_v, sed_v, sedall_v, cntmat_v,
             sem_i0, sem_i1, sem_r0, sem_r1, acc_sh):
    cid = lax.axis_index("c")
    sid = lax.axis_index("s")
    wid = cid * NTILES + sid
    sem_i = (sem_i0, sem_i1)
    sem_r = (sem_r0, sem_r1)

    zero16 = jnp.zeros((16,), jnp.float32)
    for i in range(SEG_PER_TILE):
        for q in range(D_FEAT // 16):
            rows_v[0, i, pl.ds(q * 16, 16)] = zero16
    seg0 = sid * SEG_PER_TILE
    pltpu.sync_copy(rows_v.at[0, pl.ds(0, SEG_PER_TILE)], acc_sh.at[pl.ds(seg0, SEG_PER_TILE)])

    izero16 = jnp.zeros((16,), jnp.int32)
    for r in range(2 * NUM_GRAPHS // 128):
        for q in range(8):
            sed_v[r, pl.ds(q * 16, 16)] = izero16
    iota16 = lax.iota(jnp.int32, 16)

    def _count_chunk(t, carry):
        c = t * NTILES + sid

        @pl.when(c < NCHUNKS)
        def _():
            @pl.when(c == 0)
            def _():
                idx1_v[pl.ds(0, 16)] = jnp.full((16,), -1, jnp.int32)
                pltpu.sync_copy(
                    bflat_hbm.at[pl.ds(0, CHUNK + 8)], idx1_v.at[pl.ds(8, CHUNK + 8)]
                )

            @pl.when(c > 0)
            def _():
                pltpu.sync_copy(
                    bflat_hbm.at[pl.ds(c * CHUNK - 8, CHUNK + 16)], idx1_v
                )

            for k in range(CHUNK // 16):
                cur = idx1_v[pl.ds(8 + k * 16, 16)]
                prv = idx1_v[pl.ds(7 + k * 16, 16)]
                m = cur != prv
                pos = jnp.full((16,), c * CHUNK + k * 16, jnp.int32) + iota16
                plsc.store_scatter(
                    sed_v, [cur >> 7, cur & 127], pos, mask=m
                )
                pe = prv + NUM_GRAPHS
                plsc.store_scatter(
                    sed_v, [pe >> 7, pe & 127], pos, mask=m & (prv >= 0)
                )

            @pl.when(c == NCHUNKS - 1)
            def _():
                le = idx1_v[pl.ds(8 + CHUNK - 16, 16)] + NUM_GRAPHS
                plsc.store_scatter(
                    sed_v,
                    [le >> 7, le & 127],
                    jnp.full((16,), N_NODES, jnp.int32),
                    mask=iota16 == 15,
                )

        return carry

    lax.fori_loop(0, NCHUNKS // NTILES + 1, _count_chunk, 0)

    pltpu.sync_copy(sed_v, sed_hbm.at[cid, sid])

    plsc.subcore_barrier()

    def start_gather(j, b):
        chunk = j * NWORKERS + wid
        for s in range(NSUB):
            pltpu.async_copy(
                bflat_hbm.at[pl.ds(chunk * CHUNK + s * SUB, SUB)],
                idx_v.at[b, s], sem_i[b],
            )
        pltpu.async_copy(x_hbm.at[pl.ds(chunk * CHUNK, CHUNK)], rows_v.at[b], sem_r[b])

    def wait_gather(j, b):
        chunk = j * NWORKERS + wid
        for s in range(NSUB):
            pltpu.make_async_copy(
                bflat_hbm.at[pl.ds(chunk * CHUNK + s * SUB, SUB)],
                idx_v.at[b, s], sem_i[b],
            ).wait()
        pltpu.make_async_copy(
            x_hbm.at[pl.ds(chunk * CHUNK, CHUNK)], rows_v.at[b], sem_r[b]
        ).wait()

    @pl.when(wid < NCHUNKS)
    def _():
        start_gather(0, 0)

    for j in range(NITER):
        chunk = j * NWORKERS + wid
        b = j % 2

        @pl.when(chunk < NCHUNKS)
        def _():
            wait_gather(j, b)

            @pl.when((j + 1) * NWORKERS + wid < NCHUNKS)
            def _():
                start_gather(j + 1, 1 - b)

            for s in range(NSUB):
                pltpu.sync_copy(
                    rows_v.at[b, pl.ds(s * SUB, SUB)], acc_sh.at[idx_v.at[b, s]],
                    add=True,
                )

    plsc.subcore_barrier()

    pltpu.sync_copy(acc_sh.at[pl.ds(seg0, SEG_PER_TILE)], rows_v.at[0, pl.ds(0, SEG_PER_TILE)])
    pltpu.sync_copy(
        rows_v.at[0, pl.ds(0, SEG_PER_TILE)],
        psum_hbm.at[cid, pl.ds(seg0, SEG_PER_TILE)],
    )

    pltpu.sync_copy(sed_hbm.at[cid], sedall_v)
    r0 = seg0 // 128
    er0 = NUM_GRAPHS // 128 + r0
    col0 = seg0 % 128
    for h in range(SEG_PER_TILE // 16):
        s_acc = sedall_v[0, r0, pl.ds(col0 + h * 16, 16)]
        e_acc = sedall_v[0, er0, pl.ds(col0 + h * 16, 16)]
        for k in range(1, NTILES):
            s_acc = jnp.maximum(s_acc, sedall_v[k, r0, pl.ds(col0 + h * 16, 16)])
            e_acc = jnp.maximum(e_acc, sedall_v[k, er0, pl.ds(col0 + h * 16, 16)])
        cnt_f = (e_acc - s_acc).astype(jnp.float32)
        for rr in range(16):
            cntmat_v[h * 16 + rr] = jnp.where(iota16 == rr, cnt_f, 0.0)
    pltpu.sync_copy(cntmat_v, pcnt_hbm.at[cid, pl.ds(seg0, SEG_PER_TILE)])


def _combine_body(ps_ref, pc_ref, out_ref):
    sums = ps_ref[0] + ps_ref[1]
    cnts = jnp.sum(pc_ref[0], axis=1, keepdims=True)
    cnts = jnp.maximum(cnts, 1.0)
    out_ref[...] = sums / jnp.broadcast_to(cnts, sums.shape)


@jax.jit
def _pooled(x, batch):
    mesh = plsc.VectorSubcoreMesh(core_axis_name="c", subcore_axis_name="s")
    run = functools.partial(
        pl.kernel,
        mesh=mesh,
        compiler_params=pltpu.CompilerParams(needs_layout_passes=False),
        out_type=[
            jax.ShapeDtypeStruct((NCORES, NUM_GRAPHS, D_FEAT), jnp.float32),
            jax.ShapeDtypeStruct((NCORES, NUM_GRAPHS, 16), jnp.float32),
            jax.ShapeDtypeStruct(
                (NCORES, NTILES, 2 * NUM_GRAPHS // 128, 128), jnp.int32
            ),
        ],
        scratch_types=[
            pltpu.VMEM((2, NSUB, SUB), jnp.int32),
            pltpu.VMEM((2, CHUNK, D_FEAT), jnp.float32),
            pltpu.VMEM((CHUNK + 16,), jnp.int32),
            pltpu.VMEM((2 * NUM_GRAPHS // 128, 128), jnp.int32),
            pltpu.VMEM((NTILES, 2 * NUM_GRAPHS // 128, 128), jnp.int32),
            pltpu.VMEM((SEG_PER_TILE, 16), jnp.float32),
            pltpu.SemaphoreType.DMA,
            pltpu.SemaphoreType.DMA,
            pltpu.SemaphoreType.DMA,
            pltpu.SemaphoreType.DMA,
            pltpu.VMEM_SHARED((NUM_GRAPHS, D_FEAT), jnp.float32),
        ],
    )(_sc_body)
    psum, pcnt, _ = run(x, batch)
    return pl.pallas_call(
        _combine_body,
        out_shape=jax.ShapeDtypeStruct((NUM_GRAPHS, D_FEAT), jnp.float32),
    )(psum, pcnt)


def kernel(x, batch):
    return _pooled(x, batch)


# speedup vs baseline: 1.1990x; 1.1990x over previous
import functools

import jax
import jax.numpy as jnp
from jax import lax
from jax.experimental import pallas as pl
from jax.experimental.pallas import tpu as pltpu
from jax.experimental.pallas import tpu_sc as plsc

N_NODES = 100000
D_FEAT = 128
NUM_GRAPHS = 512

CHUNK = 400
SUB = 80
NSUB = CHUNK // SUB
NCHUNKS = N_NODES // CHUNK
NTILES = 16
NCORES = 2
NWORKERS = NCORES * NTILES
NITER = (NCHUNKS + NWORKERS - 1) // NWORKERS
SEG_PER_TILE = NUM_GRAPHS // NTILES
CSPAN = 6256
CSPAN_L = N_NODES - (NTILES - 1) * CSPAN


def _sc_body(x_hbm, bflat_hbm, psum_hbm, pcnt_hbm, sed_hbm, idx_v,
             rows_v, idx1_v, sed_v, sedall_v, cntmat_v,
             sem_i0, sem_i1, sem_r0, sem_r1, sem_c, acc_sh):
    cid = lax.axis_index("c")
    sid = lax.axis_index("s")
    wid = cid * NTILES + sid
    sem_i = (sem_i0, sem_i1)
    sem_r = (sem_r0, sem_r1)

    zero16 = jnp.zeros((16,), jnp.float32)
    for i in range(SEG_PER_TILE):
        for q in range(D_FEAT // 16):
            rows_v[0, i, pl.ds(q * 16, 16)] = zero16
    seg0 = sid * SEG_PER_TILE
    pltpu.sync_copy(rows_v.at[0, pl.ds(0, SEG_PER_TILE)], acc_sh.at[pl.ds(seg0, SEG_PER_TILE)])

    izero16 = jnp.zeros((16,), jnp.int32)
    for r in range(2 * NUM_GRAPHS // 128):
        for q in range(8):
            sed_v[r, pl.ds(q * 16, 16)] = izero16
    iota16 = lax.iota(jnp.int32, 16)
    start = sid * CSPAN

    @pl.when(sid == 0)
    def _():
        idx1_v[pl.ds(0, 16)] = jnp.full((16,), -1, jnp.int32)
        pltpu.async_copy(
            bflat_hbm.at[pl.ds(0, CSPAN + 8)], idx1_v.at[pl.ds(8, CSPAN + 8)], sem_c
        )

    @pl.when((sid > 0) & (sid < NTILES - 1))
    def _():
        pltpu.async_copy(
            bflat_hbm.at[pl.ds(start - 8, CSPAN + 16)],
            idx1_v.at[pl.ds(0, CSPAN + 16)], sem_c,
        )

    @pl.when(sid == NTILES - 1)
    def _():
        pltpu.async_copy(
            bflat_hbm.at[pl.ds(start - 8, CSPAN_L + 16)],
            idx1_v.at[pl.ds(0, CSPAN_L + 16)], sem_c,
        )

    def start_gather(j, b):
        chunk = j * NWORKERS + wid
        for s in range(NSUB):
            pltpu.async_copy(
                bflat_hbm.at[pl.ds(chunk * CHUNK + s * SUB, SUB)],
                idx_v.at[b, s], sem_i[b],
            )
        pltpu.async_copy(x_hbm.at[pl.ds(chunk * CHUNK, CHUNK)], rows_v.at[b], sem_r[b])

    def wait_gather(j, b):
        chunk = j * NWORKERS + wid
        for s in range(NSUB):
            pltpu.make_async_copy(
                bflat_hbm.at[pl.ds(chunk * CHUNK + s * SUB, SUB)],
                idx_v.at[b, s], sem_i[b],
            ).wait()
        pltpu.make_async_copy(
            x_hbm.at[pl.ds(chunk * CHUNK, CHUNK)], rows_v.at[b], sem_r[b]
        ).wait()

    @pl.when(wid < NCHUNKS)
    def _():
        start_gather(0, 0)

    @pl.when(sid == 0)
    def _():
        pltpu.make_async_copy(
            bflat_hbm.at[pl.ds(0, CSPAN + 8)], idx1_v.at[pl.ds(8, CSPAN + 8)], sem_c
        ).wait()

    @pl.when((sid > 0) & (sid < NTILES - 1))
    def _():
        pltpu.make_async_copy(
            bflat_hbm.at[pl.ds(start - 8, CSPAN + 16)],
            idx1_v.at[pl.ds(0, CSPAN + 16)], sem_c,
        ).wait()

    @pl.when(sid == NTILES - 1)
    def _():
        pltpu.make_async_copy(
            bflat_hbm.at[pl.ds(start - 8, CSPAN_L + 16)],
            idx1_v.at[pl.ds(0, CSPAN_L + 16)], sem_c,
        ).wait()

    def _scan_vreg(k, carry):
        cur = idx1_v[pl.ds(8 + k * 16, 16)]
        prv = idx1_v[pl.ds(7 + k * 16, 16)]
        m = cur != prv
        pos = jnp.full((16,), start, jnp.int32) + k * 16 + iota16
        plsc.store_scatter(sed_v, [cur >> 7, cur & 127], pos, mask=m)
        pe = prv + NUM_GRAPHS
        plsc.store_scatter(sed_v, [pe >> 7, pe & 127], pos, mask=m & (prv >= 0))
        return carry

    nv = jnp.where(sid == NTILES - 1, CSPAN_L // 16, CSPAN // 16)
    lax.fori_loop(0, nv, _scan_vreg, 0)

    @pl.when(sid == NTILES - 1)
    def _():
        le = idx1_v[pl.ds(8 + CSPAN_L - 16, 16)] + NUM_GRAPHS
        plsc.store_scatter(
            sed_v,
            [le >> 7, le & 127],
            jnp.full((16,), N_NODES, jnp.int32),
            mask=iota16 == 15,
        )

    pltpu.sync_copy(sed_v, sed_hbm.at[cid, sid])

    plsc.subcore_barrier()

    for j in range(NITER):
        chunk = j * NWORKERS + wid
        b = j % 2

        @pl.when(chunk < NCHUNKS)
        def _():
            wait_gather(j, b)

            @pl.when((j + 1) * NWORKERS + wid < NCHUNKS)
            def _():
                start_gather(j + 1, 1 - b)

            for s in range(NSUB):
                pltpu.sync_copy(
                    rows_v.at[b, pl.ds(s * SUB, SUB)], acc_sh.at[idx_v.at[b, s]],
                    add=True,
                )

    plsc.subcore_barrier()

    pltpu.sync_copy(acc_sh.at[pl.ds(seg0, SEG_PER_TILE)], rows_v.at[0, pl.ds(0, SEG_PER_TILE)])
    pltpu.sync_copy(
        rows_v.at[0, pl.ds(0, SEG_PER_TILE)],
        psum_hbm.at[cid, pl.ds(seg0, SEG_PER_TILE)],
    )

    r0 = seg0 // 128
    er0 = NUM_GRAPHS // 128 + r0
    col0 = seg0 % 128
    for k in range(NTILES):
        pltpu.async_copy(sed_hbm.at[cid, k, r0], sedall_v.at[k, 0], sem_c)
        pltpu.async_copy(sed_hbm.at[cid, k, er0], sedall_v.at[k, 1], sem_c)
    for k in range(NTILES):
        pltpu.make_async_copy(sed_hbm.at[cid, k, r0], sedall_v.at[k, 0], sem_c).wait()
        pltpu.make_async_copy(sed_hbm.at[cid, k, er0], sedall_v.at[k, 1], sem_c).wait()
    for h in range(SEG_PER_TILE // 16):
        s_acc = sedall_v[0, 0, pl.ds(col0 + h * 16, 16)]
        e_acc = sedall_v[0, 1, pl.ds(col0 + h * 16, 16)]
        for k in range(1, NTILES):
            s_acc = jnp.maximum(s_acc, sedall_v[k, 0, pl.ds(col0 + h * 16, 16)])
            e_acc = jnp.maximum(e_acc, sedall_v[k, 1, pl.ds(col0 + h * 16, 16)])
        cnt_f = (e_acc - s_acc).astype(jnp.float32)
        for rr in range(16):
            cntmat_v[h * 16 + rr] = jnp.where(iota16 == rr, cnt_f, 0.0)
    pltpu.sync_copy(cntmat_v, pcnt_hbm.at[cid, pl.ds(seg0, SEG_PER_TILE)])


def _combine_body(ps_ref, pc_ref, out_ref):
    sums = ps_ref[0] + ps_ref[1]
    cnts = jnp.sum(pc_ref[0], axis=1, keepdims=True)
    cnts = jnp.maximum(cnts, 1.0)
    out_ref[...] = sums / jnp.broadcast_to(cnts, sums.shape)


@jax.jit
def _pooled(x, batch):
    mesh = plsc.VectorSubcoreMesh(core_axis_name="c", subcore_axis_name="s")
    run = functools.partial(
        pl.kernel,
        mesh=mesh,
        compiler_params=pltpu.CompilerParams(needs_layout_passes=False),
        out_type=[
            jax.ShapeDtypeStruct((NCORES, NUM_GRAPHS, D_FEAT), jnp.float32),
            jax.ShapeDtypeStruct((NCORES, NUM_GRAPHS, 16), jnp.float32),
            jax.ShapeDtypeStruct(
                (NCORES, NTILES, 2 * NUM_GRAPHS // 128, 128), jnp.int32
            ),
        ],
        scratch_types=[
            pltpu.VMEM((2, NSUB, SUB), jnp.int32),
            pltpu.VMEM((2, CHUNK, D_FEAT), jnp.float32),
            pltpu.VMEM((CSPAN + 16,), jnp.int32),
            pltpu.VMEM((2 * NUM_GRAPHS // 128, 128), jnp.int32),
            pltpu.VMEM((NTILES, 2, 128), jnp.int32),
            pltpu.VMEM((SEG_PER_TILE, 16), jnp.float32),
            pltpu.SemaphoreType.DMA,
            pltpu.SemaphoreType.DMA,
            pltpu.SemaphoreType.DMA,
            pltpu.SemaphoreType.DMA,
            pltpu.SemaphoreType.DMA,
            pltpu.VMEM_SHARED((NUM_GRAPHS, D_FEAT), jnp.float32),
        ],
    )(_sc_body)
    psum, pcnt, _ = run(x, batch)
    return pl.pallas_call(
        _combine_body,
        out_shape=jax.ShapeDtypeStruct((NUM_GRAPHS, D_FEAT), jnp.float32),
    )(psum, pcnt)


def kernel(x, batch):
    return _pooled(x, batch)


# speedup vs baseline: 1.2112x; 1.0102x over previous
import functools

import jax
import jax.numpy as jnp
from jax import lax
from jax.experimental import pallas as pl
from jax.experimental.pallas import tpu as pltpu
from jax.experimental.pallas import tpu_sc as plsc

N_NODES = 100000
D_FEAT = 128
NUM_GRAPHS = 512

CHUNK = 400
SUB = 80
NSUB = CHUNK // SUB
NCHUNKS = N_NODES // CHUNK
NTILES = 16
NCORES = 2
NWORKERS = NCORES * NTILES
NITER = (NCHUNKS + NWORKERS - 1) // NWORKERS
SEG_PER_TILE = NUM_GRAPHS // NTILES
CSPAN = 6256
CSPAN_L = N_NODES - (NTILES - 1) * CSPAN


def _sc_body(x_hbm, bflat_hbm, psum_hbm, pcnt_hbm, sed_hbm, idx_v, idx_r,
             rows_v, idx1_v, sed_v, sedall_v, cntmat_v,
             sem_i0, sem_i1, sem_r0, sem_r1, sem_c, acc_sh):
    cid = lax.axis_index("c")
    sid = lax.axis_index("s")
    wid = cid * NTILES + sid
    sem_i = (sem_i0, sem_i1)
    sem_r = (sem_r0, sem_r1)

    zero16 = jnp.zeros((16,), jnp.float32)
    for i in range(SEG_PER_TILE):
        for q in range(D_FEAT // 16):
            rows_v[0, i, pl.ds(q * 16, 16)] = zero16
    seg0 = sid * SEG_PER_TILE
    pltpu.sync_copy(rows_v.at[0, pl.ds(0, SEG_PER_TILE)], acc_sh.at[pl.ds(seg0, SEG_PER_TILE)])

    izero16 = jnp.zeros((16,), jnp.int32)
    for r in range(2 * NUM_GRAPHS // 128):
        for q in range(8):
            sed_v[r, pl.ds(q * 16, 16)] = izero16
    iota16 = lax.iota(jnp.int32, 16)
    start = sid * CSPAN

    @pl.when(sid == 0)
    def _():
        idx1_v[pl.ds(0, 16)] = jnp.full((16,), -1, jnp.int32)
        pltpu.async_copy(
            bflat_hbm.at[pl.ds(0, CSPAN + 8)], idx1_v.at[pl.ds(8, CSPAN + 8)], sem_c
        )

    @pl.when((sid > 0) & (sid < NTILES - 1))
    def _():
        pltpu.async_copy(
            bflat_hbm.at[pl.ds(start - 8, CSPAN + 16)],
            idx1_v.at[pl.ds(0, CSPAN + 16)], sem_c,
        )

    @pl.when(sid == NTILES - 1)
    def _():
        pltpu.async_copy(
            bflat_hbm.at[pl.ds(start - 8, CSPAN_L + 16)],
            idx1_v.at[pl.ds(0, CSPAN_L + 16)], sem_c,
        )

    def start_gather(j, b):
        chunk = j * NWORKERS + wid
        for s in range(3):
            pltpu.async_copy(
                bflat_hbm.at[pl.ds(chunk * CHUNK + s * 128, 128)],
                idx_v.at[b, s], sem_i[b],
            )
        pltpu.async_copy(
            bflat_hbm.at[pl.ds(chunk * CHUNK + 384, 16)], idx_r.at[b], sem_i[b]
        )
        pltpu.async_copy(x_hbm.at[pl.ds(chunk * CHUNK, CHUNK)], rows_v.at[b], sem_r[b])

    def wait_gather(j, b):
        chunk = j * NWORKERS + wid
        for s in range(3):
            pltpu.make_async_copy(
                bflat_hbm.at[pl.ds(chunk * CHUNK + s * 128, 128)],
                idx_v.at[b, s], sem_i[b],
            ).wait()
        pltpu.make_async_copy(
            bflat_hbm.at[pl.ds(chunk * CHUNK + 384, 16)], idx_r.at[b], sem_i[b]
        ).wait()
        pltpu.make_async_copy(
            x_hbm.at[pl.ds(chunk * CHUNK, CHUNK)], rows_v.at[b], sem_r[b]
        ).wait()

    @pl.when(wid < NCHUNKS)
    def _():
        start_gather(0, 0)

    @pl.when(sid == 0)
    def _():
        pltpu.make_async_copy(
            bflat_hbm.at[pl.ds(0, CSPAN + 8)], idx1_v.at[pl.ds(8, CSPAN + 8)], sem_c
        ).wait()

    @pl.when((sid > 0) & (sid < NTILES - 1))
    def _():
        pltpu.make_async_copy(
            bflat_hbm.at[pl.ds(start - 8, CSPAN + 16)],
            idx1_v.at[pl.ds(0, CSPAN + 16)], sem_c,
        ).wait()

    @pl.when(sid == NTILES - 1)
    def _():
        pltpu.make_async_copy(
            bflat_hbm.at[pl.ds(start - 8, CSPAN_L + 16)],
            idx1_v.at[pl.ds(0, CSPAN_L + 16)], sem_c,
        ).wait()

    def _scan_vreg(k, carry):
        cur = idx1_v[pl.ds(8 + k * 16, 16)]
        prv = idx1_v[pl.ds(7 + k * 16, 16)]
        m = cur != prv
        pos = jnp.full((16,), start, jnp.int32) + k * 16 + iota16
        plsc.store_scatter(sed_v, [cur >> 7, cur & 127], pos, mask=m)
        pe = prv + NUM_GRAPHS
        plsc.store_scatter(sed_v, [pe >> 7, pe & 127], pos, mask=m & (prv >= 0))
        return carry

    nv = jnp.where(sid == NTILES - 1, CSPAN_L // 16, CSPAN // 16)
    lax.fori_loop(0, nv, _scan_vreg, 0)

    @pl.when(sid == NTILES - 1)
    def _():
        le = idx1_v[pl.ds(8 + CSPAN_L - 16, 16)] + NUM_GRAPHS
        plsc.store_scatter(
            sed_v,
            [le >> 7, le & 127],
            jnp.full((16,), N_NODES, jnp.int32),
            mask=iota16 == 15,
        )

    pltpu.sync_copy(sed_v, sed_hbm.at[cid, sid])

    plsc.subcore_barrier()

    for j in range(NITER):
        chunk = j * NWORKERS + wid
        b = j % 2

        @pl.when(chunk < NCHUNKS)
        def _():
            wait_gather(j, b)

            @pl.when((j + 1) * NWORKERS + wid < NCHUNKS)
            def _():
                start_gather(j + 1, 1 - b)

            for s in range(3):
                pltpu.sync_copy(
                    rows_v.at[b, pl.ds(s * 128, 128)], acc_sh.at[idx_v.at[b, s]],
                    add=True,
                )
            pltpu.sync_copy(
                rows_v.at[b, pl.ds(384, 16)], acc_sh.at[idx_r.at[b]], add=True
            )

    plsc.subcore_barrier()

    pltpu.sync_copy(acc_sh.at[pl.ds(seg0, SEG_PER_TILE)], rows_v.at[0, pl.ds(0, SEG_PER_TILE)])
    pltpu.sync_copy(
        rows_v.at[0, pl.ds(0, SEG_PER_TILE)],
        psum_hbm.at[cid, pl.ds(seg0, SEG_PER_TILE)],
    )

    r0 = seg0 // 128
    er0 = NUM_GRAPHS // 128 + r0
    col0 = seg0 % 128
    for k in range(NTILES):
        pltpu.async_copy(sed_hbm.at[cid, k, r0], sedall_v.at[k, 0], sem_c)
        pltpu.async_copy(sed_hbm.at[cid, k, er0], sedall_v.at[k, 1], sem_c)
    for k in range(NTILES):
        pltpu.make_async_copy(sed_hbm.at[cid, k, r0], sedall_v.at[k, 0], sem_c).wait()
        pltpu.make_async_copy(sed_hbm.at[cid, k, er0], sedall_v.at[k, 1], sem_c).wait()
    for h in range(SEG_PER_TILE // 16):
        s_acc = sedall_v[0, 0, pl.ds(col0 + h * 16, 16)]
        e_acc = sedall_v[0, 1, pl.ds(col0 + h * 16, 16)]
        for k in range(1, NTILES):
            s_acc = jnp.maximum(s_acc, sedall_v[k, 0, pl.ds(col0 + h * 16, 16)])
            e_acc = jnp.maximum(e_acc, sedall_v[k, 1, pl.ds(col0 + h * 16, 16)])
        cnt_f = (e_acc - s_acc).astype(jnp.float32)
        for rr in range(16):
            cntmat_v[h * 16 + rr] = jnp.where(iota16 == rr, cnt_f, 0.0)
    pltpu.sync_copy(cntmat_v, pcnt_hbm.at[cid, pl.ds(seg0, SEG_PER_TILE)])


def _combine_body(ps_ref, pc_ref, out_ref):
    sums = ps_ref[0] + ps_ref[1]
    cnts = jnp.sum(pc_ref[0], axis=1, keepdims=True)
    cnts = jnp.maximum(cnts, 1.0)
    out_ref[...] = sums / jnp.broadcast_to(cnts, sums.shape)


@jax.jit
def _pooled(x, batch):
    mesh = plsc.VectorSubcoreMesh(core_axis_name="c", subcore_axis_name="s")
    run = functools.partial(
        pl.kernel,
        mesh=mesh,
        compiler_params=pltpu.CompilerParams(needs_layout_passes=False),
        out_type=[
            jax.ShapeDtypeStruct((NCORES, NUM_GRAPHS, D_FEAT), jnp.float32),
            jax.ShapeDtypeStruct((NCORES, NUM_GRAPHS, 16), jnp.float32),
            jax.ShapeDtypeStruct(
                (NCORES, NTILES, 2 * NUM_GRAPHS // 128, 128), jnp.int32
            ),
        ],
        scratch_types=[
            pltpu.VMEM((2, 3, 128), jnp.int32),
            pltpu.VMEM((2, 16), jnp.int32),
            pltpu.VMEM((2, CHUNK, D_FEAT), jnp.float32),
            pltpu.VMEM((CSPAN + 16,), jnp.int32),
            pltpu.VMEM((2 * NUM_GRAPHS // 128, 128), jnp.int32),
            pltpu.VMEM((NTILES, 2, 128), jnp.int32),
            pltpu.VMEM((SEG_PER_TILE, 16), jnp.float32),
            pltpu.SemaphoreType.DMA,
            pltpu.SemaphoreType.DMA,
            pltpu.SemaphoreType.DMA,
            pltpu.SemaphoreType.DMA,
            pltpu.SemaphoreType.DMA,
            pltpu.VMEM_SHARED((NUM_GRAPHS, D_FEAT), jnp.float32),
        ],
    )(_sc_body)
    psum, pcnt, _ = run(x, batch)
    return pl.pallas_call(
        _combine_body,
        out_shape=jax.ShapeDtypeStruct((NUM_GRAPHS, D_FEAT), jnp.float32),
    )(psum, pcnt)


def kernel(x, batch):
    return _pooled(x, batch)
